# baseline (speedup 1.0000x reference)
import functools

import jax
import jax.numpy as jnp
from jax import lax
from jax.experimental import pallas as pl
from jax.experimental.pallas import tpu as pltpu
from jax.experimental.pallas import tpu_sc as plsc

N = 10000
E = 320000
D = 16
NC = 2
NS = 16
TILES = NC * NS
CW = 125
CH = E // (TILES * CW)
CHN = E // (NS * CW)
NP = 10240
RPT = NP // NS
RSTG = 160
UNR = 8

_f32 = jnp.float32
_MESH = plsc.VectorSubcoreMesh(
    core_axis_name="c", subcore_axis_name="s", num_cores=NC, num_subcores=NS)


def _row_loop(nrows, body_row):
  def body(i, carry):
    for u in range(UNR):
      body_row(i * UNR + u)
    return carry
  lax.fori_loop(0, nrows // UNR, body, 0)
  for r in range(nrows - nrows % UNR, nrows):
    body_row(r)


def _fill_rows(ref, nrows, value):
  def fill(r):
    ref[r, :] = jnp.full((D,), value, _f32)
  _row_loop(nrows, fill)


def _rsqrt_vec(v):
  i = lax.bitcast_convert_type(v, jnp.int32)
  y = lax.bitcast_convert_type(
      jnp.full((D,), 0x5F3759DF, jnp.int32) - (i >> 1), _f32)
  for _ in range(2):
    y = y * (1.5 - 0.5 * v * y * y)
  return y


def _zero_acc(stage, acc, s):
  for k in range(RPT // RSTG):
    pltpu.sync_copy(stage, acc.at[pl.ds(s * RPT + k * RSTG, RSTG)])


def _read_acc(stage, acc, out, s, stage2=None, w0=None, w1=None):
  if stage2 is None:
    for k in range(RPT // RSTG):
      rows = pl.ds(s * RPT + k * RSTG, RSTG)
      pltpu.sync_copy(acc.at[rows], stage)
      pltpu.sync_copy(stage, out.at[rows])
    return
  bufs = (stage, stage2)
  sems = (w0, w1)
  nch = RPT // RSTG
  for k in range(nch):
    rows = pl.ds(s * RPT + k * RSTG, RSTG)
    b = k % 2
    if k >= 2:
      pltpu.make_async_copy(bufs[b], out.at[rows], sems[b]).wait()
    pltpu.sync_copy(acc.at[rows], bufs[b])
    pltpu.async_copy(bufs[b], out.at[rows], sems[b])
  for k in range(max(nch - 2, 0), nch):
    b = k % 2
    rows = pl.ds(s * RPT + k * RSTG, RSTG)
    pltpu.make_async_copy(bufs[b], out.at[rows], sems[b]).wait()


def _edge_loop(tbl, acc, idx_s, idx_d, rows_bufs, gsems, ssems):
  for b in range(4):
    pltpu.async_copy(tbl.at[idx_s.at[b]], rows_bufs[b], gsems[b])

  def body(j4, carry):
    for u in range(4):
      j = j4 * 4 + u
      pltpu.make_async_copy(tbl.at[idx_s.at[0]], rows_bufs[u],
                            gsems[u]).wait()
      pltpu.async_copy(rows_bufs[u], acc.at[idx_d.at[j]], ssems[u], add=True)
      b2 = (u + 2) % 4
      jj = j - 2

      @pl.when((jj >= 0) & (jj + 4 < CH))
      def _():
        pltpu.make_async_copy(rows_bufs[b2], acc.at[idx_d.at[0]],
                              ssems[b2]).wait()
        pltpu.async_copy(tbl.at[idx_s.at[jj + 4]], rows_bufs[b2], gsems[b2])
    return carry
  lax.fori_loop(0, CH // 4, body, 0)
  for b in range(4):
    pltpu.make_async_copy(rows_bufs[b], acc.at[idx_d.at[0]], ssems[b]).wait()


@functools.partial(
    pl.kernel,
    mesh=_MESH,
    compiler_params=pltpu.CompilerParams(use_tc_tiling_on_sc=False),
    out_type=[jax.ShapeDtypeStruct((NP, D), _f32)] * 2,
    scratch_types=[
        pltpu.VMEM((CHN, CW), jnp.int32),
        pltpu.VMEM((CW, D), _f32),
        pltpu.VMEM((RSTG, D), _f32),
        pltpu.VMEM((RSTG, D), _f32),
        pltpu.VMEM_SHARED((NP, D), _f32),
        pltpu.SemaphoreType.DMA,
        pltpu.SemaphoreType.DMA,
        pltpu.SemaphoreType.DMA,
        pltpu.SemaphoreType.DMA,
    ],
)
def _sc_norms(src_hbm, dst_hbm, ns, nd,
              idx, ones_v, stage, stage2, acc, s0, s1, s2, s3):
  c = lax.axis_index("c")
  s = lax.axis_index("s")
  sems = (s0, s1, s2, s3)

  @pl.when(c == 0)
  def _():
    pltpu.async_copy(src_hbm.at[pl.ds(s * CHN, CHN)], idx, s0)

  @pl.when(c == 1)
  def _():
    pltpu.async_copy(dst_hbm.at[pl.ds(s * CHN, CHN)], idx, s0)

  _fill_rows(ones_v, CW, 1.0)
  _fill_rows(stage, RSTG, 0.0)
  _zero_acc(stage, acc, s)
  pltpu.make_async_copy(src_hbm.at[pl.ds(s * CHN, CHN)], idx, s0).wait()
  plsc.subcore_barrier()

  def body(j4, carry):
    for b in range(4):
      j = j4 * 4 + b

      @pl.when(j4 > 0)
      def _():
        pltpu.make_async_copy(ones_v, acc.at[idx.at[0]], sems[b]).wait()
      pltpu.async_copy(ones_v, acc.at[idx.at[j]], sems[b], add=True)
    return carry
  lax.fori_loop(0, CHN // 4, body, 0)
  for b in range(4):
    pltpu.make_async_copy(ones_v, acc.at[idx.at[0]], sems[b]).wait()
  plsc.subcore_barrier()

  def emit(out):
    bufs = (stage, stage2)
    nch = RPT // RSTG
    for k in range(nch):
      rows = pl.ds(s * RPT + k * RSTG, RSTG)
      b = k % 2
      if k >= 2:
        pltpu.make_async_copy(bufs[b], out.at[rows], sems[b]).wait()
      pltpu.sync_copy(acc.at[rows], bufs[b])

      def norm_row(r):
        bufs[b][r, :] = _rsqrt_vec(jnp.maximum(bufs[b][r, :], 1.0))
      _row_loop(RSTG, norm_row)
      pltpu.async_copy(bufs[b], out.at[rows], sems[b])
    for k in range(max(nch - 2, 0), nch):
      rows = pl.ds(s * RPT + k * RSTG, RSTG)
      pltpu.make_async_copy(bufs[k % 2], out.at[rows], sems[k % 2]).wait()

  @pl.when(c == 0)
  def _():
    emit(ns)

  @pl.when(c == 1)
  def _():
    emit(nd)


_PASS_SCRATCH = [
    pltpu.VMEM((CH, CW), jnp.int32),
    pltpu.VMEM((CH, CW), jnp.int32),
    pltpu.VMEM((CW, D), _f32),
    pltpu.VMEM((CW, D), _f32),
    pltpu.VMEM((CW, D), _f32),
    pltpu.VMEM((CW, D), _f32),
    pltpu.VMEM((RSTG, D), _f32),
    pltpu.VMEM((RSTG, D), _f32),
    pltpu.VMEM_SHARED((NP, D), _f32),
    pltpu.SemaphoreType.DMA,
    pltpu.SemaphoreType.DMA,
    pltpu.SemaphoreType.DMA,
    pltpu.SemaphoreType.DMA,
    pltpu.SemaphoreType.DMA,
    pltpu.SemaphoreType.DMA,
    pltpu.SemaphoreType.DMA,
    pltpu.SemaphoreType.DMA,
]


@functools.partial(
    pl.kernel,
    mesh=_MESH,
    compiler_params=pltpu.CompilerParams(use_tc_tiling_on_sc=False),
    out_type=[jax.ShapeDtypeStruct((NP, D), _f32)] * 3,
    scratch_types=_PASS_SCRATCH,
)
def _sc_pass1(xw, ns, src_hbm, dst_hbm, p0, p1, tbl,
              idx_s, idx_d, r0, r1, r2, r3, bufa, bufb, acc,
              g0, g1, g2, g3, s0, s1, s2, s3):
  c = lax.axis_index("c")
  s = lax.axis_index("s")
  g = c * NS + s

  pltpu.async_copy(src_hbm.at[pl.ds(g * CH, CH)], idx_s, s0)
  pltpu.async_copy(dst_hbm.at[pl.ds(g * CH, CH)], idx_d, s1)
  for k in range(RPT // RSTG):
    rows = pl.ds(s * RPT + k * RSTG, RSTG)
    pltpu.async_copy(ns.at[rows], bufa, g0)
    pltpu.async_copy(xw.at[rows], bufb, g1)
    pltpu.make_async_copy(ns.at[rows], bufa, g0).wait()
    pltpu.make_async_copy(xw.at[rows], bufb, g1).wait()

    def scale_row(r):
      bufb[r, :] = bufb[r, :] * bufa[r, :]
    _row_loop(RSTG, scale_row)
    pltpu.sync_copy(bufb, tbl.at[rows])

  _fill_rows(bufa, RSTG, 0.0)
  _zero_acc(bufa, acc, s)
  pltpu.make_async_copy(src_hbm.at[pl.ds(g * CH, CH)], idx_s, s0).wait()
  pltpu.make_async_copy(dst_hbm.at[pl.ds(g * CH, CH)], idx_d, s1).wait()
  plsc.subcore_barrier()

  _edge_loop(tbl, acc, idx_s, idx_d, (r0, r1, r2, r3),
             (g0, g1, g2, g3), (s0, s1, s2, s3))
  plsc.subcore_barrier()

  @pl.when(c == 0)
  def _():
    _read_acc(bufa, acc, p0, s, bufb, g0, g1)

  @pl.when(c == 1)
  def _():
    _read_acc(bufa, acc, p1, s, bufb, g0, g1)


@functools.partial(
    pl.kernel,
    mesh=_MESH,
    compiler_params=pltpu.CompilerParams(use_tc_tiling_on_sc=False),
    out_type=[jax.ShapeDtypeStruct((NP, D), _f32)] * 3,
    scratch_types=_PASS_SCRATCH + [
        pltpu.VMEM((RSTG, D), _f32),
        pltpu.VMEM((RSTG, D), _f32),
        pltpu.VMEM((D,), _f32),
    ],
)
def _sc_pass2(p10, p11, nd, ns, b1, src_hbm, dst_hbm, p0, p1, tbl,
              idx_s, idx_d, r0, r1, r2, r3, bufa, bufb, acc,
              g0, g1, g2, g3, s0, s1, s2, s3, bufc, bufe, b1v):
  c = lax.axis_index("c")
  s = lax.axis_index("s")
  g = c * NS + s

  pltpu.async_copy(src_hbm.at[pl.ds(g * CH, CH)], idx_s, s0)
  pltpu.async_copy(dst_hbm.at[pl.ds(g * CH, CH)], idx_d, s1)
  pltpu.sync_copy(b1, b1v)
  for k in range(RPT // RSTG):
    rows = pl.ds(s * RPT + k * RSTG, RSTG)
    pltpu.async_copy(p10.at[rows], bufa, g0)
    pltpu.async_copy(p11.at[rows], bufb, g1)
    pltpu.async_copy(nd.at[rows], bufc, g2)
    pltpu.async_copy(ns.at[rows], bufe, g3)
    pltpu.make_async_copy(p10.at[rows], bufa, g0).wait()
    pltpu.make_async_copy(p11.at[rows], bufb, g1).wait()
    pltpu.make_async_copy(nd.at[rows], bufc, g2).wait()
    pltpu.make_async_copy(ns.at[rows], bufe, g3).wait()

    def one_row(r):
      h1 = jnp.maximum(
          (bufa[r, :] + bufb[r, :]) * bufc[r, :] + b1v[...], 0.0)
      bufa[r, :] = h1 * bufe[r, :]
    _row_loop(RSTG, one_row)
    pltpu.sync_copy(bufa, tbl.at[rows])

  _fill_rows(bufa, RSTG, 0.0)
  _zero_acc(bufa, acc, s)
  pltpu.make_async_copy(src_hbm.at[pl.ds(g * CH, CH)], idx_s, s0).wait()
  pltpu.make_async_copy(dst_hbm.at[pl.ds(g * CH, CH)], idx_d, s1).wait()
  plsc.subcore_barrier()

  _edge_loop(tbl, acc, idx_s, idx_d, (r0, r1, r2, r3),
             (g0, g1, g2, g3), (s0, s1, s2, s3))
  plsc.subcore_barrier()

  @pl.when(c == 0)
  def _():
    _read_acc(bufa, acc, p0, s, bufb, g0, g1)

  @pl.when(c == 1)
  def _():
    _read_acc(bufa, acc, p1, s, bufb, g0, g1)


_BLK = 1000
_GRID = N // _BLK


def _row_spec():
  return pl.BlockSpec((_BLK, D), lambda i: (i, 0))


def _xw_body(x, w1, xw_o):
  xw_o[...] = jnp.dot(x[...], w1[...], preferred_element_type=_f32)


def _tc_xw(x, w1):
  return pl.pallas_call(
      _xw_body,
      grid=(_GRID,),
      in_specs=[
          pl.BlockSpec((_BLK, 128), lambda i: (i, 0)),
          pl.BlockSpec((128, D), lambda i: (0, 0)),
      ],
      out_specs=_row_spec(),
      out_shape=jax.ShapeDtypeStruct((NP, D), _f32),
  )(x, w1)


def _final_body(p0, p1, nd, w2, b2, wl, bl, out_o, mx):
  agg = (p0[...] + p1[...]) * nd[...]
  h2 = jnp.dot(agg, w2[...], preferred_element_type=_f32) + b2[...]
  m = jnp.max(h2, axis=0, keepdims=True)
  i = pl.program_id(0)

  @pl.when(i == 0)
  def _():
    mx[...] = m

  @pl.when(i > 0)
  def _():
    mx[...] = jnp.maximum(mx[...], m)

  @pl.when(i == _GRID - 1)
  def _():
    out_o[...] = jnp.dot(mx[...], wl[...], preferred_element_type=_f32) + bl[...]


def _tc_final(p0, p1, nd, w2, b2, wl, bl):
  n_classes = wl.shape[1]
  return pl.pallas_call(
      _final_body,
      grid=(_GRID,),
      in_specs=[
          _row_spec(), _row_spec(), _row_spec(),
          pl.BlockSpec((D, D), lambda i: (0, 0)),
          pl.BlockSpec((D,), lambda i: (0,)),
          pl.BlockSpec((D, n_classes), lambda i: (0, 0)),
          pl.BlockSpec((n_classes,), lambda i: (0,)),
      ],
      out_specs=pl.BlockSpec((1, n_classes), lambda i: (0, 0)),
      out_shape=jax.ShapeDtypeStruct((1, n_classes), _f32),
      scratch_shapes=[pltpu.VMEM((1, D), _f32)],
  )(p0, p1, nd, w2, b2, wl, bl)


def kernel(x, edge_index, W1, b1, W2, b2, Wl, bl):
  src2 = edge_index[0].reshape(E // CW, CW)
  dst2 = edge_index[1].reshape(E // CW, CW)

  xw = _tc_xw(x, W1)
  ns, nd = _sc_norms(src2, dst2)
  p10, p11, _h0 = _sc_pass1(xw, ns, src2, dst2)
  p20, p21, _h1 = _sc_pass2(p10, p11, nd, ns, b1, src2, dst2)
  return _tc_final(p20, p21, nd, W2, b2, Wl, bl)


# speedup vs baseline: 1.1581x; 1.1581x over previous
import functools

import jax
import jax.numpy as jnp
from jax import lax
from jax.experimental import pallas as pl
from jax.experimental.pallas import tpu as pltpu
from jax.experimental.pallas import tpu_sc as plsc

N = 10000
E = 320000
D = 16
NC = 2
NS = 16
TILES = NC * NS
CW = 125
CH = E // (TILES * CW)
CHN = E // (NS * CW)
NP = 10240
RPT = NP // NS
RSTG = 160
UNR = 8

_f32 = jnp.float32
_MESH = plsc.VectorSubcoreMesh(
    core_axis_name="c", subcore_axis_name="s", num_cores=NC, num_subcores=NS)


def _row_loop(nrows, body_row):
  def body(i, carry):
    for u in range(UNR):
      body_row(i * UNR + u)
    return carry
  lax.fori_loop(0, nrows // UNR, body, 0)
  for r in range(nrows - nrows % UNR, nrows):
    body_row(r)


def _fill_rows(ref, nrows, value):
  def fill(r):
    ref[r, :] = jnp.full((D,), value, _f32)
  _row_loop(nrows, fill)


def _rsqrt_vec(v):
  i = lax.bitcast_convert_type(v, jnp.int32)
  y = lax.bitcast_convert_type(
      jnp.full((D,), 0x5F3759DF, jnp.int32) - (i >> 1), _f32)
  for _ in range(2):
    y = y * (1.5 - 0.5 * v * y * y)
  return y


def _zero_acc(stage, acc, s):
  for k in range(RPT // RSTG):
    pltpu.sync_copy(stage, acc.at[pl.ds(s * RPT + k * RSTG, RSTG)])


def _read_acc(stage, acc, out, s, stage2=None, w0=None, w1=None):
  if stage2 is None:
    for k in range(RPT // RSTG):
      rows = pl.ds(s * RPT + k * RSTG, RSTG)
      pltpu.sync_copy(acc.at[rows], stage)
      pltpu.sync_copy(stage, out.at[rows])
    return
  bufs = (stage, stage2)
  sems = (w0, w1)
  nch = RPT // RSTG
  for k in range(nch):
    rows = pl.ds(s * RPT + k * RSTG, RSTG)
    b = k % 2
    if k >= 2:
      pltpu.make_async_copy(bufs[b], out.at[rows], sems[b]).wait()
    pltpu.sync_copy(acc.at[rows], bufs[b])
    pltpu.async_copy(bufs[b], out.at[rows], sems[b])
  for k in range(max(nch - 2, 0), nch):
    b = k % 2
    rows = pl.ds(s * RPT + k * RSTG, RSTG)
    pltpu.make_async_copy(bufs[b], out.at[rows], sems[b]).wait()


def _edge_loop(tbl, acc, idx_s, idx_d, rows_bufs, gsems, ssems):
  for b in range(4):
    pltpu.async_copy(tbl.at[idx_s.at[b]], rows_bufs[b], gsems[b])

  def body(j4, carry):
    for b in range(4):
      j = j4 * 4 + b
      pltpu.make_async_copy(tbl.at[idx_s.at[0]], rows_bufs[b],
                            gsems[b]).wait()
      pltpu.async_copy(rows_bufs[b], acc.at[idx_d.at[j]], ssems[b], add=True)

      @pl.when(j + 4 < CH)
      def _():
        pltpu.make_async_copy(rows_bufs[b], acc.at[idx_d.at[0]],
                              ssems[b]).wait()
        pltpu.async_copy(tbl.at[idx_s.at[j + 4]], rows_bufs[b], gsems[b])
    return carry
  lax.fori_loop(0, CH // 4, body, 0)
  for b in range(4):
    pltpu.make_async_copy(rows_bufs[b], acc.at[idx_d.at[0]], ssems[b]).wait()


@functools.partial(
    pl.kernel,
    mesh=_MESH,
    compiler_params=pltpu.CompilerParams(use_tc_tiling_on_sc=False),
    out_type=[jax.ShapeDtypeStruct((NP, D), _f32)] * 2,
    scratch_types=[
        pltpu.VMEM((CHN, CW), jnp.int32),
        pltpu.VMEM((CW, D), _f32),
        pltpu.VMEM((RSTG, D), _f32),
        pltpu.VMEM((RSTG, D), _f32),
        pltpu.VMEM_SHARED((NP, D), _f32),
        pltpu.SemaphoreType.DMA,
        pltpu.SemaphoreType.DMA,
        pltpu.SemaphoreType.DMA,
        pltpu.SemaphoreType.DMA,
    ],
)
def _sc_norms(src_hbm, dst_hbm, ns, nd,
              idx, ones_v, stage, stage2, acc, s0, s1, s2, s3):
  c = lax.axis_index("c")
  s = lax.axis_index("s")
  sems = (s0, s1, s2, s3)

  @pl.when(c == 0)
  def _():
    pltpu.async_copy(src_hbm.at[pl.ds(s * CHN, CHN)], idx, s0)

  @pl.when(c == 1)
  def _():
    pltpu.async_copy(dst_hbm.at[pl.ds(s * CHN, CHN)], idx, s0)

  _fill_rows(ones_v, CW, 1.0)
  _fill_rows(stage, RSTG, 0.0)
  _zero_acc(stage, acc, s)
  pltpu.make_async_copy(src_hbm.at[pl.ds(s * CHN, CHN)], idx, s0).wait()
  plsc.subcore_barrier()

  def body(j4, carry):
    for b in range(4):
      j = j4 * 4 + b

      @pl.when(j4 > 0)
      def _():
        pltpu.make_async_copy(ones_v, acc.at[idx.at[0]], sems[b]).wait()
      pltpu.async_copy(ones_v, acc.at[idx.at[j]], sems[b], add=True)
    return carry
  lax.fori_loop(0, CHN // 4, body, 0)
  for b in range(4):
    pltpu.make_async_copy(ones_v, acc.at[idx.at[0]], sems[b]).wait()
  plsc.subcore_barrier()

  def emit(out):
    bufs = (stage, stage2)
    nch = RPT // RSTG
    for k in range(nch):
      rows = pl.ds(s * RPT + k * RSTG, RSTG)
      b = k % 2
      if k >= 2:
        pltpu.make_async_copy(bufs[b], out.at[rows], sems[b]).wait()
      pltpu.sync_copy(acc.at[rows], bufs[b])

      def norm_row(r):
        bufs[b][r, :] = _rsqrt_vec(jnp.maximum(bufs[b][r, :], 1.0))
      _row_loop(RSTG, norm_row)
      pltpu.async_copy(bufs[b], out.at[rows], sems[b])
    for k in range(max(nch - 2, 0), nch):
      rows = pl.ds(s * RPT + k * RSTG, RSTG)
      pltpu.make_async_copy(bufs[k % 2], out.at[rows], sems[k % 2]).wait()

  @pl.when(c == 0)
  def _():
    emit(ns)

  @pl.when(c == 1)
  def _():
    emit(nd)


_PASS_SCRATCH = [
    pltpu.VMEM((CH, CW), jnp.int32),
    pltpu.VMEM((CH, CW), jnp.int32),
    pltpu.VMEM((CW, D), _f32),
    pltpu.VMEM((CW, D), _f32),
    pltpu.VMEM((CW, D), _f32),
    pltpu.VMEM((CW, D), _f32),
    pltpu.VMEM((RSTG, D), _f32),
    pltpu.VMEM((RSTG, D), _f32),
    pltpu.VMEM_SHARED((NP, D), _f32),
    pltpu.SemaphoreType.DMA,
    pltpu.SemaphoreType.DMA,
    pltpu.SemaphoreType.DMA,
    pltpu.SemaphoreType.DMA,
    pltpu.SemaphoreType.DMA,
    pltpu.SemaphoreType.DMA,
    pltpu.SemaphoreType.DMA,
    pltpu.SemaphoreType.DMA,
]


@functools.partial(
    pl.kernel,
    mesh=_MESH,
    compiler_params=pltpu.CompilerParams(use_tc_tiling_on_sc=False),
    out_type=[jax.ShapeDtypeStruct((NP, D), _f32)] * 3,
    scratch_types=_PASS_SCRATCH,
)
def _sc_pass1(xw, ns, src_hbm, dst_hbm, p0, p1, tbl,
              idx_s, idx_d, r0, r1, r2, r3, bufa, bufb, acc,
              g0, g1, g2, g3, s0, s1, s2, s3):
  c = lax.axis_index("c")
  s = lax.axis_index("s")
  g = c * NS + s

  pltpu.async_copy(src_hbm.at[pl.ds(g * CH, CH)], idx_s, s0)
  pltpu.async_copy(dst_hbm.at[pl.ds(g * CH, CH)], idx_d, s1)
  for k in range(RPT // RSTG):
    rows = pl.ds(s * RPT + k * RSTG, RSTG)
    pltpu.async_copy(ns.at[rows], bufa, g0)
    pltpu.async_copy(xw.at[rows], bufb, g1)
    pltpu.make_async_copy(ns.at[rows], bufa, g0).wait()
    pltpu.make_async_copy(xw.at[rows], bufb, g1).wait()

    def scale_row(r):
      bufb[r, :] = bufb[r, :] * bufa[r, :]
    _row_loop(RSTG, scale_row)
    pltpu.sync_copy(bufb, tbl.at[rows])

  _fill_rows(bufa, RSTG, 0.0)
  _zero_acc(bufa, acc, s)
  pltpu.make_async_copy(src_hbm.at[pl.ds(g * CH, CH)], idx_s, s0).wait()
  pltpu.make_async_copy(dst_hbm.at[pl.ds(g * CH, CH)], idx_d, s1).wait()
  plsc.subcore_barrier()

  _edge_loop(tbl, acc, idx_s, idx_d, (r0, r1, r2, r3),
             (g0, g1, g2, g3), (s0, s1, s2, s3))
  plsc.subcore_barrier()

  @pl.when(c == 0)
  def _():
    _read_acc(bufa, acc, p0, s, bufb, g0, g1)

  @pl.when(c == 1)
  def _():
    _read_acc(bufa, acc, p1, s, bufb, g0, g1)


@functools.partial(
    pl.kernel,
    mesh=_MESH,
    compiler_params=pltpu.CompilerParams(use_tc_tiling_on_sc=False),
    out_type=[jax.ShapeDtypeStruct((NP, D), _f32)] * 3,
    scratch_types=_PASS_SCRATCH + [
        pltpu.VMEM((RSTG, D), _f32),
        pltpu.VMEM((RSTG, D), _f32),
        pltpu.VMEM((D,), _f32),
    ],
)
def _sc_pass2(p10, p11, nd, ns, b1, src_hbm, dst_hbm, p0, p1, tbl,
              idx_s, idx_d, r0, r1, r2, r3, bufa, bufb, acc,
              g0, g1, g2, g3, s0, s1, s2, s3, bufc, bufe, b1v):
  c = lax.axis_index("c")
  s = lax.axis_index("s")
  g = c * NS + s

  pltpu.async_copy(src_hbm.at[pl.ds(g * CH, CH)], idx_s, s0)
  pltpu.async_copy(dst_hbm.at[pl.ds(g * CH, CH)], idx_d, s1)
  pltpu.sync_copy(b1, b1v)
  for k in range(RPT // RSTG):
    rows = pl.ds(s * RPT + k * RSTG, RSTG)
    pltpu.async_copy(p10.at[rows], bufa, g0)
    pltpu.async_copy(p11.at[rows], bufb, g1)
    pltpu.async_copy(nd.at[rows], bufc, g2)
    pltpu.async_copy(ns.at[rows], bufe, g3)
    pltpu.make_async_copy(p10.at[rows], bufa, g0).wait()
    pltpu.make_async_copy(p11.at[rows], bufb, g1).wait()
    pltpu.make_async_copy(nd.at[rows], bufc, g2).wait()
    pltpu.make_async_copy(ns.at[rows], bufe, g3).wait()

    def one_row(r):
      h1 = jnp.maximum(
          (bufa[r, :] + bufb[r, :]) * bufc[r, :] + b1v[...], 0.0)
      bufa[r, :] = h1 * bufe[r, :]
    _row_loop(RSTG, one_row)
    pltpu.sync_copy(bufa, tbl.at[rows])

  _fill_rows(bufa, RSTG, 0.0)
  _zero_acc(bufa, acc, s)
  pltpu.make_async_copy(src_hbm.at[pl.ds(g * CH, CH)], idx_s, s0).wait()
  pltpu.make_async_copy(dst_hbm.at[pl.ds(g * CH, CH)], idx_d, s1).wait()
  plsc.subcore_barrier()

  _edge_loop(tbl, acc, idx_s, idx_d, (r0, r1, r2, r3),
             (g0, g1, g2, g3), (s0, s1, s2, s3))
  plsc.subcore_barrier()

  @pl.when(c == 0)
  def _():
    _read_acc(bufa, acc, p0, s, bufb, g0, g1)

  @pl.when(c == 1)
  def _():
    _read_acc(bufa, acc, p1, s, bufb, g0, g1)


_BLK = 1000
_GRID = N // _BLK


def _row_spec():
  return pl.BlockSpec((_BLK, D), lambda i: (i, 0))


def _xw_body(x, w1, xw_o):
  xw_o[...] = jnp.dot(x[...], w1[...], preferred_element_type=_f32)


def _tc_xw(x, w1):
  return pl.pallas_call(
      _xw_body,
      grid=(_GRID,),
      in_specs=[
          pl.BlockSpec((_BLK, 128), lambda i: (i, 0)),
          pl.BlockSpec((128, D), lambda i: (0, 0)),
      ],
      out_specs=_row_spec(),
      out_shape=jax.ShapeDtypeStruct((NP, D), _f32),
  )(x, w1)


def _final_body(p0, p1, nd, w2, b2, wl, bl, out_o, mx):
  agg = (p0[...] + p1[...]) * nd[...]
  h2 = jnp.dot(agg, w2[...], preferred_element_type=_f32) + b2[...]
  m = jnp.max(h2, axis=0, keepdims=True)
  i = pl.program_id(0)

  @pl.when(i == 0)
  def _():
    mx[...] = m

  @pl.when(i > 0)
  def _():
    mx[...] = jnp.maximum(mx[...], m)

  @pl.when(i == _GRID - 1)
  def _():
    out_o[...] = jnp.dot(mx[...], wl[...], preferred_element_type=_f32) + bl[...]


def _tc_final(p0, p1, nd, w2, b2, wl, bl):
  n_classes = wl.shape[1]
  return pl.pallas_call(
      _final_body,
      grid=(_GRID,),
      in_specs=[
          _row_spec(), _row_spec(), _row_spec(),
          pl.BlockSpec((D, D), lambda i: (0, 0)),
          pl.BlockSpec((D,), lambda i: (0,)),
          pl.BlockSpec((D, n_classes), lambda i: (0, 0)),
          pl.BlockSpec((n_classes,), lambda i: (0,)),
      ],
      out_specs=pl.BlockSpec((1, n_classes), lambda i: (0, 0)),
      out_shape=jax.ShapeDtypeStruct((1, n_classes), _f32),
      scratch_shapes=[pltpu.VMEM((1, D), _f32)],
  )(p0, p1, nd, w2, b2, wl, bl)


def kernel(x, edge_index, W1, b1, W2, b2, Wl, bl):
  src2 = edge_index[0].reshape(E // CW, CW)
  dst2 = edge_index[1].reshape(E // CW, CW)

  xw = _tc_xw(x, W1)
  ns, nd = _sc_norms(src2, dst2)
  p10, p11, _h0 = _sc_pass1(xw, ns, src2, dst2)
  p20, p21, _h1 = _sc_pass2(p10, p11, nd, ns, b1, src2, dst2)
  return _tc_final(p20, p21, nd, W2, b2, Wl, bl)
